# Initial kernel scaffold; baseline (speedup 1.0000x reference)
import jax
import jax.numpy as jnp
from jax.experimental import pallas as pl


def kernel(x, thresholds, data, hash_values, input_order, mask, bias):
    raise NotImplementedError("write your pallas kernel here")


# speedup vs baseline: 17.3576x; 17.3576x over previous
import functools

import jax
import jax.numpy as jnp
from jax import lax
from jax.experimental import pallas as pl
from jax.experimental.pallas import tpu as pltpu
from jax.experimental.pallas import tpu_sc as plsc

B = 4096
NI = 64
BPI = 32
UI = 32
UE = 65536
NH = 2
NCLS = 10
F = 64
IB = NI * BPI

BB = 512
PE = 2048

NW = 32
ROWS_W = B // NW
DRAIN = 16


def _hash_body(x_ref, thr_ref, hv_ref, oq_ref, out_ref):
    oq = oq_ref[0:1, :]
    src = oq // BPI
    tcol = oq % BPI
    i_iota = lax.broadcasted_iota(jnp.int32, (NI, IB), 0)
    sel_i = (i_iota == src).astype(jnp.float32)
    t_iota = lax.broadcasted_iota(jnp.int32, (BPI, IB), 0)
    sel_t = (t_iota == tcol).astype(jnp.float32)
    xq = lax.dot_general(x_ref[...], sel_i, (((1,), (0,)), ((), ())),
                         precision=lax.Precision.HIGHEST,
                         preferred_element_type=jnp.float32)
    w2 = lax.dot_general(thr_ref[...], sel_t, (((1,), (0,)), ((), ())),
                         precision=lax.Precision.HIGHEST,
                         preferred_element_type=jnp.float32)
    thrq = jnp.sum(sel_i * w2, axis=0, keepdims=True)
    bits = xq >= thrq
    q_iota = lax.broadcasted_iota(jnp.int32, (UI, IB), 1)
    j_iota = lax.broadcasted_iota(jnp.int32, (UI, IB), 0)
    sel_j = (j_iota == q_iota // F).astype(jnp.float32)
    f_off = lax.broadcasted_iota(jnp.int32, (x_ref.shape[0], F), 1) * UE
    for h in range(NH):
        coef = lax.dot_general(hv_ref[h:h + 1, :], sel_j,
                               (((1,), (0,)), ((), ())),
                               precision=lax.Precision.HIGHEST,
                               preferred_element_type=jnp.float32)
        coef_i = coef.astype(jnp.int32)
        sel = jnp.where(bits, coef_i, 0)
        w = IB
        while w > F:
            w //= 2
            sel = sel[:, :w] ^ sel[:, w:2 * w]
        out_ref[h, :, :] = sel + f_off


def _hash_indices(x, thresholds, hv_pad, oq_pad):
    return pl.pallas_call(
        _hash_body,
        grid=(B // BB,),
        in_specs=[
            pl.BlockSpec((BB, NI), lambda i: (i, 0)),
            pl.BlockSpec((NI, BPI), lambda i: (0, 0)),
            pl.BlockSpec((8, BPI), lambda i: (0, 0)),
            pl.BlockSpec((8, IB), lambda i: (0, 0)),
        ],
        out_specs=pl.BlockSpec((NH, BB, F), lambda i: (0, i, 0)),
        out_shape=jax.ShapeDtypeStruct((NH, B, F), jnp.int32),
    )(x, thresholds, hv_pad, oq_pad)


def _pack_body(d_ref, out_ref):
    w = jnp.zeros((F, PE), jnp.int32)
    for c in range(NCLS):
        w = w + (d_ref[c] >= 0).astype(jnp.int32) * (1 << c)
    out_ref[...] = w


def _pack_table(data):
    return pl.pallas_call(
        _pack_body,
        grid=(UE // PE,),
        in_specs=[pl.BlockSpec((NCLS, F, PE), lambda e: (0, 0, e))],
        out_specs=pl.BlockSpec((F, PE), lambda e: (0, e)),
        out_shape=jax.ShapeDtypeStruct((F, UE), jnp.int32),
    )(data)


def _sc_gather_body(idx_hbm, table_hbm, out_hbm, idx_v, out_v, sem):
    wid = lax.axis_index("s") * 2 + lax.axis_index("c")
    base = wid * ROWS_W
    pltpu.sync_copy(idx_hbm.at[pl.ds(base, ROWS_W)], idx_v)

    def group(g, carry):
        descs = []
        for b in range(DRAIN):
            r = g * DRAIN + b
            descs.append(
                pltpu.async_copy(table_hbm.at[idx_v.at[r]], out_v.at[r], sem))
        for d in descs:
            d.wait()
        return carry

    lax.fori_loop(0, ROWS_W // DRAIN, group, 0)
    pltpu.sync_copy(out_v, out_hbm.at[pl.ds(base, ROWS_W)])


def _sc_gather(idx2d, table_flat):
    fn = functools.partial(
        pl.kernel,
        mesh=plsc.VectorSubcoreMesh(core_axis_name="c", subcore_axis_name="s"),
        out_type=jax.ShapeDtypeStruct((B, NH * F), jnp.int32),
        scratch_types=[
            pltpu.VMEM((ROWS_W, 128), jnp.int32),
            pltpu.VMEM((ROWS_W, 128), jnp.int32),
            pltpu.SemaphoreType.DMA,
        ],
    )(_sc_gather_body)
    return fn(idx2d, table_flat)


def _reduce_body(g_ref, mask_ref, bias_ref, out_ref):
    w = g_ref[0] & g_ref[1]
    cols = []
    for c in range(NCLS):
        plane = ((w >> c) & 1).astype(jnp.float32)
        m = mask_ref[c:c + 1, :]
        cols.append(jnp.sum(plane * m, axis=1, keepdims=True))
    res = jnp.concatenate(cols, axis=1)
    out_ref[...] = res + bias_ref[0:1, :NCLS]


def _reduce(gathered, mask_pad, bias_pad):
    return pl.pallas_call(
        _reduce_body,
        grid=(1,),
        in_specs=[
            pl.BlockSpec((NH, B, F), lambda i: (0, 0, 0)),
            pl.BlockSpec((16, F), lambda i: (0, 0)),
            pl.BlockSpec((8, 16), lambda i: (0, 0)),
        ],
        out_specs=pl.BlockSpec((B, NCLS), lambda i: (0, 0)),
        out_shape=jax.ShapeDtypeStruct((B, NCLS), jnp.float32),
    )(gathered, mask_pad, bias_pad)


def kernel(x, thresholds, data, hash_values, input_order, mask, bias):
    oq = input_order.astype(jnp.int32).reshape(F, UI).T.reshape(1, IB)
    oq_pad = jnp.pad(oq, ((0, 7), (0, 0)))
    hv_pad = jnp.pad(hash_values.astype(jnp.float32), ((0, 8 - NH), (0, 0)))
    mask_pad = jnp.pad(mask, ((0, 16 - NCLS), (0, 0)))
    bias_pad = jnp.pad(bias.reshape(1, NCLS), ((0, 7), (0, 16 - NCLS)))

    gidx = _hash_indices(x, thresholds, hv_pad, oq_pad)
    table = _pack_table(data)
    gathered = _sc_gather(gidx.reshape(B, NH * F), table.reshape(F * UE))
    return _reduce(gathered.reshape(NH, B, F), mask_pad, bias_pad)
